# baseline (speedup 1.0000x reference)
import functools

import jax
import jax.numpy as jnp
from jax import lax
from jax.experimental import pallas as pl
from jax.experimental.pallas import tpu as pltpu
from jax.experimental.pallas import tpu_sc as plsc

N = 10000
E = 320000
H = 128
G = 256
C = 10
L = 3

NC = 2
NS = 16
NW = NC * NS
EDGES_PER_TILE = E // NS
CHUNK = 64
NCHUNKS = 320
EPT_PAD = NCHUNKS * CHUNK
SCRAP = N
NP = 10240
ROWS_PER_TILE = NP // NS
PCHUNK = 80
PBASE = 640
NEG = -3.0e38


def _seg_sum_sc(x_cat, src_cat, dst_cat, zeros_n):
    mesh = plsc.VectorSubcoreMesh(core_axis_name="c", subcore_axis_name="s")

    @functools.partial(
        pl.kernel,
        out_type=jax.ShapeDtypeStruct((NC, NP, H), jnp.float32),
        mesh=mesh,
        scratch_types=(
            [pltpu.VMEM((CHUNK,), jnp.int32)] * 4
            + [pltpu.VMEM((CHUNK, H), jnp.float32)] * 2
            + [pltpu.VMEM_SHARED((NP, H), jnp.float32)]
            + [pltpu.SemaphoreType.DMA] * 6
        ),
    )
    def seg_sum_kernel(x_hbm, src_hbm, dst_hbm, zeros_hbm, out_hbm,
                       sv0, sv1, dv0, dv1, row0, row1, acc_sh,
                       si0, si1, sd0, sd1, sr0, sr1):
        src_v = (sv0, sv1)
        dst_v = (dv0, dv1)
        rows = (row0, row1)
        sem_s = (si0, si1)
        sem_d = (sd0, sd1)
        sem_r = (sr0, sr1)
        c = lax.axis_index("c")
        s = lax.axis_index("s")
        r0 = s * ROWS_PER_TILE
        pltpu.sync_copy(zeros_hbm.at[pl.ds(r0, ROWS_PER_TILE)],
                        acc_sh.at[pl.ds(r0, ROWS_PER_TILE)])
        plsc.subcore_barrier()
        base = (c * NS + s) * EPT_PAD

        def load_idx(i, b):
            off = base + i * CHUNK
            pltpu.async_copy(src_hbm.at[pl.ds(off, CHUNK)], src_v[b],
                             sem_s[b])
            pltpu.async_copy(dst_hbm.at[pl.ds(off, CHUNK)], dst_v[b],
                             sem_d[b])

        last = NCHUNKS - 1
        load_idx(0, 0)
        load_idx(1, 1)
        pltpu.make_async_copy(src_hbm.at[pl.ds(base, CHUNK)], sv0,
                              si0).wait()
        pltpu.async_copy(x_hbm.at[sv0], row0, sr0)

        def pair(g, carry):
            for b in range(2):
                i = 2 * g + b
                ob = 1 - b
                pltpu.make_async_copy(x_hbm.at[src_v[b]], rows[b],
                                      sem_r[b]).wait()
                pltpu.make_async_copy(src_hbm.at[pl.ds(base, CHUNK)],
                                      src_v[ob], sem_s[ob]).wait()
                pltpu.async_copy(x_hbm.at[src_v[ob]], rows[ob], sem_r[ob])
                pltpu.make_async_copy(dst_hbm.at[pl.ds(base, CHUNK)],
                                      dst_v[b], sem_d[b]).wait()
                pltpu.sync_copy(rows[b], acc_sh.at[dst_v[b]], add=True)
                load_idx(lax.min(i + 2, last), b)
            return carry

        lax.fori_loop(0, NCHUNKS // 2, pair, 0)
        pltpu.make_async_copy(src_hbm.at[pl.ds(base, CHUNK)], sv1, si1).wait()
        pltpu.make_async_copy(dst_hbm.at[pl.ds(base, CHUNK)], dv0, sd0).wait()
        pltpu.make_async_copy(dst_hbm.at[pl.ds(base, CHUNK)], dv1, sd1).wait()
        pltpu.make_async_copy(x_hbm.at[sv0], row0, sr0).wait()
        plsc.subcore_barrier()
        pltpu.sync_copy(acc_sh.at[pl.ds(r0, ROWS_PER_TILE)],
                        out_hbm.at[c, pl.ds(r0, ROWS_PER_TILE)])

    return seg_sum_kernel(x_cat, src_cat, dst_cat, zeros_n)


def _pool_sc(x_cat, ids_cat, zeros_n, neg_gh):
    mesh = plsc.VectorSubcoreMesh(core_axis_name="c", subcore_axis_name="s")

    @functools.partial(
        pl.kernel,
        out_type=(
            jax.ShapeDtypeStruct((NW, G, H), jnp.float32),
            jax.ShapeDtypeStruct((NW, G, H), jnp.float32),
            jax.ShapeDtypeStruct((NW, G, 16), jnp.float32),
        ),
        mesh=mesh,
        scratch_types=[
            pltpu.VMEM((PCHUNK,), jnp.int32),
            pltpu.VMEM((PCHUNK, H), jnp.float32),
            pltpu.VMEM((G, H), jnp.float32),
            pltpu.VMEM((G, H), jnp.float32),
            pltpu.VMEM((G, 16), jnp.float32),
        ],
    )
    def pool_kernel(x_hbm, ids_hbm, zeros_hbm, neg_hbm,
                    osum_hbm, omax_hbm, ocnt_hbm,
                    ids_v, rows_v, sum_acc, max_acc, cnt_acc):
        c = lax.axis_index("c")
        s = lax.axis_index("s")
        pltpu.sync_copy(zeros_hbm.at[pl.ds(0, G)], sum_acc)
        pltpu.sync_copy(neg_hbm, max_acc)

        def zero_cnt(i, carry):
            cnt_acc[i, :] = jnp.zeros((16,), jnp.float32)
            return carry

        lax.fori_loop(0, G, zero_cnt, 0)
        base = c * N + s * PBASE
        nch = jnp.where(s < NS - 1, PBASE // PCHUNK, (N - 15 * PBASE) // PCHUNK)

        def chunk_body(k, carry):
            row0 = base + k * PCHUNK
            pltpu.sync_copy(ids_hbm.at[pl.ds(row0, PCHUNK)], ids_v)
            pltpu.sync_copy(x_hbm.at[pl.ds(row0, PCHUNK)], rows_v)

            def grp_body(q, carry2):
                idvec = ids_v[pl.ds(q * 16, 16)]
                for r in range(16):
                    g = idvec[r]
                    i = q * 16 + r
                    for kk in range(H // 16):
                        sl = pl.ds(kk * 16, 16)
                        rv = rows_v[i, sl]
                        sum_acc[g, sl] = sum_acc[g, sl] + rv
                        max_acc[g, sl] = jnp.maximum(max_acc[g, sl], rv)
                    cnt_acc[g, :] = cnt_acc[g, :] + 1.0
                return carry2

            lax.fori_loop(0, PCHUNK // 16, grp_body, 0)
            return carry

        lax.fori_loop(0, nch, chunk_body, 0)
        w = c * NS + s
        pltpu.sync_copy(sum_acc, osum_hbm.at[w])
        pltpu.sync_copy(max_acc, omax_hbm.at[w])
        pltpu.sync_copy(cnt_acc, ocnt_hbm.at[w])

    return pool_kernel(x_cat, ids_cat, zeros_n, neg_gh)


def _dense_body(m_ref, x_ref, wr_s_ref, br_s_ref, wo_s_ref,
                wr_v_ref, br_v_ref, wo_v_ref,
                gv_ref, bv_ref, gs_ref, bs_ref, o_ref):
    xv = x_ref[0:N, :]
    xs = x_ref[N:2 * N, :]
    msg_s = m_ref[0, 0:N, :]
    msg_v = m_ref[1, 0:N, :]
    f32 = jnp.float32
    out_s = (jnp.dot(msg_s, wr_s_ref[...], preferred_element_type=f32)
             + br_s_ref[...]
             + jnp.dot(xs, wo_s_ref[...], preferred_element_type=f32))
    out_v = (jnp.dot(msg_v, wr_v_ref[...], preferred_element_type=f32)
             + br_v_ref[...]
             + jnp.dot(xv, wo_v_ref[...], preferred_element_type=f32))
    out_s = jnp.maximum(out_s, 0.0)
    out_v = jnp.maximum(out_v, 0.0)

    def bn(x, g, b):
        m = jnp.mean(x, axis=0, keepdims=True)
        v = jnp.mean((x - m) ** 2, axis=0, keepdims=True)
        return g * (x - m) / jnp.sqrt(v + 1e-5) + b

    o_ref[0:N, :] = bn(out_v, gv_ref[...], bv_ref[...])
    o_ref[N:2 * N, :] = bn(out_s, gs_ref[...], bs_ref[...])


def _combine_body(s1, m1, c1, s2, m2, c2, s3, m3, c3, w_ref, b_ref, o_ref):
    readout = jnp.zeros((G, 2 * H), jnp.float32)
    for sum_ref, max_ref, cnt_ref in ((s1, m1, c1), (s2, m2, c2), (s3, m3, c3)):
        sum_v = jnp.sum(sum_ref[0:NS], axis=0)
        sum_s = jnp.sum(sum_ref[NS:NW], axis=0)
        cnt_v = jnp.sum(cnt_ref[0:NS], axis=0)[:, 0:1]
        cnt_s = jnp.sum(cnt_ref[NS:NW], axis=0)[:, 0:1]
        mean_pool = (sum_v / jnp.maximum(cnt_v, 1.0)
                     + sum_s / jnp.maximum(cnt_s, 1.0))
        max_v = jnp.max(max_ref[0:NS], axis=0)
        max_s = jnp.max(max_ref[NS:NW], axis=0)
        max_pool = (jnp.where(cnt_v > 0.0, max_v, 0.0)
                    + jnp.where(cnt_s > 0.0, max_s, 0.0))
        readout = readout + jnp.concatenate([mean_pool, max_pool], axis=1)
    o_ref[...] = (jnp.dot(readout, w_ref[...], preferred_element_type=jnp.float32)
                  + b_ref[...])


def kernel(x_visit, x_service, edge_index_vs, edge_index_sv, batch_visit, batch_service,
           Wrel_vs, brel_vs, Wroot_vs, Wrel_sv, brel_sv, Wroot_sv,
           bn_g_visit, bn_b_visit, bn_g_service, bn_b_service, lin_W, lin_b):
    def pad_edges(e, fill):
        e2 = jnp.pad(e.reshape(NS, EDGES_PER_TILE),
                     ((0, 0), (0, EPT_PAD - EDGES_PER_TILE)),
                     constant_values=fill)
        return e2.reshape(-1)

    src_cat = jnp.concatenate([pad_edges(edge_index_vs[0], 0),
                               pad_edges(edge_index_sv[0] + N, 0)])
    dst_cat = jnp.concatenate([pad_edges(edge_index_vs[1], SCRAP),
                               pad_edges(edge_index_sv[1], SCRAP)])
    ids_cat = jnp.concatenate([batch_visit, batch_service])
    zeros_n = jnp.zeros((NP, H), jnp.float32)
    neg_gh = jnp.full((G, H), NEG, jnp.float32)

    dense = pl.pallas_call(
        _dense_body,
        out_shape=jax.ShapeDtypeStruct((2 * N, H), jnp.float32),
    )
    x_cat = jnp.concatenate([x_visit, x_service], axis=0)
    pools = []
    for l in range(L):
        msg = _seg_sum_sc(x_cat, src_cat, dst_cat, zeros_n)
        x_cat = dense(msg, x_cat,
                      Wrel_vs[l], brel_vs[l], Wroot_vs[l],
                      Wrel_sv[l], brel_sv[l], Wroot_sv[l],
                      bn_g_visit, bn_b_visit, bn_g_service, bn_b_service)
        pools.append(_pool_sc(x_cat, ids_cat, zeros_n, neg_gh))

    combine = pl.pallas_call(
        _combine_body,
        out_shape=jax.ShapeDtypeStruct((G, C), jnp.float32),
    )
    (s1, m1, c1), (s2, m2, c2), (s3, m3, c3) = pools
    return combine(s1, m1, c1, s2, m2, c2, s3, m3, c3, lin_W, lin_b)


# speedup vs baseline: 2.7668x; 2.7668x over previous
import functools

import jax
import jax.numpy as jnp
from jax import lax
from jax.experimental import pallas as pl
from jax.experimental.pallas import tpu as pltpu
from jax.experimental.pallas import tpu_sc as plsc

N = 10000
E = 320000
H = 128
G = 256
C = 10
L = 3

NC = 2
NS = 16
NW = NC * NS
EDGES_PER_TILE = E // NS
CHUNK = 80
NCHUNKS = EDGES_PER_TILE // CHUNK
NP = 10240
ROWS_PER_TILE = NP // NS
PCHUNK = 80
PBASE = 640
NEG = -3.0e38


def _seg_sum_sc(x_cat, src_cat, dst_cat, zeros_n):
    mesh = plsc.VectorSubcoreMesh(core_axis_name="c", subcore_axis_name="s")

    @functools.partial(
        pl.kernel,
        out_type=jax.ShapeDtypeStruct((NC, NP, H), jnp.float32),
        mesh=mesh,
        scratch_types=(
            [pltpu.VMEM((CHUNK,), jnp.int32)] * 4
            + [pltpu.VMEM((CHUNK, H), jnp.float32)] * 2
            + [pltpu.VMEM_SHARED((NP, H), jnp.float32)]
            + [pltpu.SemaphoreType.DMA] * 6
        ),
    )
    def seg_sum_kernel(x_hbm, src_hbm, dst_hbm, zeros_hbm, out_hbm,
                       sv0, sv1, dv0, dv1, row0, row1, acc_sh,
                       si0, si1, sd0, sd1, sr0, sr1):
        src_v = (sv0, sv1)
        dst_v = (dv0, dv1)
        rows = (row0, row1)
        sem_s = (si0, si1)
        sem_d = (sd0, sd1)
        sem_r = (sr0, sr1)
        c = lax.axis_index("c")
        s = lax.axis_index("s")
        r0 = s * ROWS_PER_TILE
        pltpu.sync_copy(zeros_hbm.at[pl.ds(r0, ROWS_PER_TILE)],
                        acc_sh.at[pl.ds(r0, ROWS_PER_TILE)])
        plsc.subcore_barrier()
        base = c * E + s * EDGES_PER_TILE

        def load_idx(i, b):
            off = base + i * CHUNK
            pltpu.async_copy(src_hbm.at[pl.ds(off, CHUNK)], src_v[b],
                             sem_s[b])
            pltpu.async_copy(dst_hbm.at[pl.ds(off, CHUNK)], dst_v[b],
                             sem_d[b])

        last = NCHUNKS - 1
        load_idx(0, 0)
        load_idx(1, 1)
        pltpu.make_async_copy(src_hbm.at[pl.ds(base, CHUNK)], sv0,
                              si0).wait()
        pltpu.async_copy(x_hbm.at[sv0], row0, sr0)

        def pair(g, carry):
            for b in range(2):
                i = 2 * g + b
                ob = 1 - b
                pltpu.make_async_copy(x_hbm.at[src_v[b]], rows[b],
                                      sem_r[b]).wait()
                pltpu.make_async_copy(src_hbm.at[pl.ds(base, CHUNK)],
                                      src_v[ob], sem_s[ob]).wait()
                pltpu.async_copy(x_hbm.at[src_v[ob]], rows[ob], sem_r[ob])
                pltpu.make_async_copy(dst_hbm.at[pl.ds(base, CHUNK)],
                                      dst_v[b], sem_d[b]).wait()
                pltpu.sync_copy(rows[b], acc_sh.at[dst_v[b]], add=True)
                load_idx(lax.min(i + 2, last), b)
            return carry

        lax.fori_loop(0, NCHUNKS // 2, pair, 0)
        pltpu.make_async_copy(src_hbm.at[pl.ds(base, CHUNK)], sv1, si1).wait()
        pltpu.make_async_copy(dst_hbm.at[pl.ds(base, CHUNK)], dv0, sd0).wait()
        pltpu.make_async_copy(dst_hbm.at[pl.ds(base, CHUNK)], dv1, sd1).wait()
        pltpu.make_async_copy(x_hbm.at[sv0], row0, sr0).wait()
        plsc.subcore_barrier()
        pltpu.sync_copy(acc_sh.at[pl.ds(r0, ROWS_PER_TILE)],
                        out_hbm.at[c, pl.ds(r0, ROWS_PER_TILE)])

    return seg_sum_kernel(x_cat, src_cat, dst_cat, zeros_n)


def _pool_sc(x_cat, ids_cat, zeros_n, neg_gh):
    mesh = plsc.VectorSubcoreMesh(core_axis_name="c", subcore_axis_name="s")

    @functools.partial(
        pl.kernel,
        out_type=(
            jax.ShapeDtypeStruct((NW, G, H), jnp.float32),
            jax.ShapeDtypeStruct((NW, G, H), jnp.float32),
            jax.ShapeDtypeStruct((NW, G, 16), jnp.float32),
        ),
        mesh=mesh,
        scratch_types=[
            pltpu.VMEM((PCHUNK,), jnp.int32),
            pltpu.VMEM((PCHUNK, H), jnp.float32),
            pltpu.VMEM((G, H), jnp.float32),
            pltpu.VMEM((G, H), jnp.float32),
            pltpu.VMEM((G, 16), jnp.float32),
        ],
    )
    def pool_kernel(x_hbm, ids_hbm, zeros_hbm, neg_hbm,
                    osum_hbm, omax_hbm, ocnt_hbm,
                    ids_v, rows_v, sum_acc, max_acc, cnt_acc):
        c = lax.axis_index("c")
        s = lax.axis_index("s")
        pltpu.sync_copy(zeros_hbm.at[pl.ds(0, G)], sum_acc)
        pltpu.sync_copy(neg_hbm, max_acc)

        def zero_cnt(i, carry):
            cnt_acc[i, :] = jnp.zeros((16,), jnp.float32)
            return carry

        lax.fori_loop(0, G, zero_cnt, 0)
        base = c * N + s * PBASE
        nch = jnp.where(s < NS - 1, PBASE // PCHUNK, (N - 15 * PBASE) // PCHUNK)

        def chunk_body(k, carry):
            row0 = base + k * PCHUNK
            pltpu.sync_copy(ids_hbm.at[pl.ds(row0, PCHUNK)], ids_v)
            pltpu.sync_copy(x_hbm.at[pl.ds(row0, PCHUNK)], rows_v)

            def grp_body(q, carry2):
                idvec = ids_v[pl.ds(q * 16, 16)]
                for r in range(16):
                    g = idvec[r]
                    i = q * 16 + r
                    for kk in range(H // 16):
                        sl = pl.ds(kk * 16, 16)
                        rv = rows_v[i, sl]
                        sum_acc[g, sl] = sum_acc[g, sl] + rv
                        max_acc[g, sl] = jnp.maximum(max_acc[g, sl], rv)
                    cnt_acc[g, :] = cnt_acc[g, :] + 1.0
                return carry2

            lax.fori_loop(0, PCHUNK // 16, grp_body, 0)
            return carry

        lax.fori_loop(0, nch, chunk_body, 0)
        w = c * NS + s
        pltpu.sync_copy(sum_acc, osum_hbm.at[w])
        pltpu.sync_copy(max_acc, omax_hbm.at[w])
        pltpu.sync_copy(cnt_acc, ocnt_hbm.at[w])

    return pool_kernel(x_cat, ids_cat, zeros_n, neg_gh)


def _dense_body(m_ref, x_ref, wr_s_ref, br_s_ref, wo_s_ref,
                wr_v_ref, br_v_ref, wo_v_ref,
                gv_ref, bv_ref, gs_ref, bs_ref, o_ref):
    xv = x_ref[0:N, :]
    xs = x_ref[N:2 * N, :]
    msg_s = m_ref[0, 0:N, :]
    msg_v = m_ref[1, 0:N, :]
    f32 = jnp.float32
    out_s = (jnp.dot(msg_s, wr_s_ref[...], preferred_element_type=f32)
             + br_s_ref[...]
             + jnp.dot(xs, wo_s_ref[...], preferred_element_type=f32))
    out_v = (jnp.dot(msg_v, wr_v_ref[...], preferred_element_type=f32)
             + br_v_ref[...]
             + jnp.dot(xv, wo_v_ref[...], preferred_element_type=f32))
    out_s = jnp.maximum(out_s, 0.0)
    out_v = jnp.maximum(out_v, 0.0)

    def bn(x, g, b):
        m = jnp.mean(x, axis=0, keepdims=True)
        v = jnp.mean((x - m) ** 2, axis=0, keepdims=True)
        return g * (x - m) / jnp.sqrt(v + 1e-5) + b

    o_ref[0:N, :] = bn(out_v, gv_ref[...], bv_ref[...])
    o_ref[N:2 * N, :] = bn(out_s, gs_ref[...], bs_ref[...])


def _combine_body(s1, m1, c1, s2, m2, c2, s3, m3, c3, w_ref, b_ref, o_ref):
    readout = jnp.zeros((G, 2 * H), jnp.float32)
    for sum_ref, max_ref, cnt_ref in ((s1, m1, c1), (s2, m2, c2), (s3, m3, c3)):
        sum_v = jnp.sum(sum_ref[0:NS], axis=0)
        sum_s = jnp.sum(sum_ref[NS:NW], axis=0)
        cnt_v = jnp.sum(cnt_ref[0:NS], axis=0)[:, 0:1]
        cnt_s = jnp.sum(cnt_ref[NS:NW], axis=0)[:, 0:1]
        mean_pool = (sum_v / jnp.maximum(cnt_v, 1.0)
                     + sum_s / jnp.maximum(cnt_s, 1.0))
        max_v = jnp.max(max_ref[0:NS], axis=0)
        max_s = jnp.max(max_ref[NS:NW], axis=0)
        max_pool = (jnp.where(cnt_v > 0.0, max_v, 0.0)
                    + jnp.where(cnt_s > 0.0, max_s, 0.0))
        readout = readout + jnp.concatenate([mean_pool, max_pool], axis=1)
    o_ref[...] = (jnp.dot(readout, w_ref[...], preferred_element_type=jnp.float32)
                  + b_ref[...])


def kernel(x_visit, x_service, edge_index_vs, edge_index_sv, batch_visit, batch_service,
           Wrel_vs, brel_vs, Wroot_vs, Wrel_sv, brel_sv, Wroot_sv,
           bn_g_visit, bn_b_visit, bn_g_service, bn_b_service, lin_W, lin_b):
    src_cat = jnp.concatenate([edge_index_vs[0], edge_index_sv[0] + N])
    dst_cat = jnp.concatenate([edge_index_vs[1], edge_index_sv[1]])
    ids_cat = jnp.concatenate([batch_visit, batch_service])
    zeros_n = jnp.zeros((NP, H), jnp.float32)
    neg_gh = jnp.full((G, H), NEG, jnp.float32)

    dense = pl.pallas_call(
        _dense_body,
        out_shape=jax.ShapeDtypeStruct((2 * N, H), jnp.float32),
    )
    x_cat = jnp.concatenate([x_visit, x_service], axis=0)
    pools = []
    for l in range(L):
        msg = _seg_sum_sc(x_cat, src_cat, dst_cat, zeros_n)
        x_cat = dense(msg, x_cat,
                      Wrel_vs[l], brel_vs[l], Wroot_vs[l],
                      Wrel_sv[l], brel_sv[l], Wroot_sv[l],
                      bn_g_visit, bn_b_visit, bn_g_service, bn_b_service)
        pools.append(_pool_sc(x_cat, ids_cat, zeros_n, neg_gh))

    combine = pl.pallas_call(
        _combine_body,
        out_shape=jax.ShapeDtypeStruct((G, C), jnp.float32),
    )
    (s1, m1, c1), (s2, m2, c2), (s3, m3, c3) = pools
    return combine(s1, m1, c1, s2, m2, c2, s3, m3, c3, lin_W, lin_b)
